# baseline (speedup 1.0000x reference)
import jax
import jax.numpy as jnp
from jax.experimental import pallas as pl
from jax.experimental.pallas import tpu as pltpu


def _fused_heads_kernel(x_ref, wb_ref, bb_ref, wi_ref, bi_ref, od_ref, oi_ref):
    x = x_ref[...]
    od_ref[...] = (
        jnp.dot(x, wb_ref[...], preferred_element_type=jnp.float32) + bb_ref[...]
    )
    oi_ref[...] = (
        jnp.dot(x, wi_ref[...], preferred_element_type=jnp.float32) + bi_ref[...]
    )


def kernel(x, W_bbox, b_bbox, W_iou, b_iou):
    if x.ndim > 2:
        x = x.reshape(x.shape[0], -1)
    N, D = x.shape
    C = W_bbox.shape[1]
    bb2 = b_bbox.reshape(1, C)
    bi2 = b_iou.reshape(1, 1)

    BN = 2000
    grid = (N // BN,)

    out_shapes = (
        jax.ShapeDtypeStruct((N, C), jnp.float32),
        jax.ShapeDtypeStruct((N, 1), jnp.float32),
    )
    od, oi = pl.pallas_call(
        _fused_heads_kernel,
        grid=grid,
        in_specs=[
            pl.BlockSpec((BN, D), lambda i: (i, 0)),
            pl.BlockSpec((D, C), lambda i: (0, 0)),
            pl.BlockSpec((1, C), lambda i: (0, 0)),
            pl.BlockSpec((D, 1), lambda i: (0, 0)),
            pl.BlockSpec((1, 1), lambda i: (0, 0)),
        ],
        out_specs=(
            pl.BlockSpec((BN, C), lambda i: (i, 0)),
            pl.BlockSpec((BN, 1), lambda i: (i, 0)),
        ),
        out_shape=out_shapes,
        compiler_params=pltpu.CompilerParams(
            dimension_semantics=("arbitrary",),
        ),
    )(x, W_bbox, bb2, W_iou, bi2)
    return (od, oi)
